# baseline (speedup 1.0000x reference)
import functools

import jax
import jax.numpy as jnp
from jax import lax
from jax.experimental import pallas as pl
from jax.experimental.pallas import tpu as pltpu
from jax.experimental.pallas import tpu_sc as plsc

_N = 5000
_L = 16
_NW = 32
_NPAD = 5120
_NCHUNK = _NPAD // _L
_CAP = _NPAD + 128
_OB = 80

_BIG = 1e30


def _scalar_i32(v):
    return jnp.max(v)


def _scalar_f32(v):
    return jnp.max(v)


def _nms_body(xl_h, yl_h, xh_h, yh_h, sc_h, lab_h, par_h, out_h,
              xl, yl, xh, yh, sc, lab, parv,
              cands, rows,
              cxs, cys, cxe, cye, css, cpo, car,
              outv, myidx, sem):
    cid = lax.axis_index("c")
    sid = lax.axis_index("s")
    wid = sid * 2 + cid
    pltpu.sync_copy(xl_h, xl)
    pltpu.sync_copy(yl_h, yl)
    pltpu.sync_copy(xh_h, xh)
    pltpu.sync_copy(yh_h, yh)
    pltpu.sync_copy(sc_h, sc)
    pltpu.sync_copy(lab_h, lab)
    pltpu.sync_copy(par_h, parv)

    lane = lax.iota(jnp.int32, _L)
    lane0 = lane == 0
    zero16 = jnp.zeros((_L,), jnp.int32)
    xmin = _scalar_f32(plsc.load_gather(parv, [zero16]))
    sw = _scalar_f32(plsc.load_gather(parv, [zero16 + 1]))
    wmax = _scalar_f32(plsc.load_gather(parv, [zero16 + 2]))

    wf = wid.astype(jnp.float32)
    row_lo = jnp.where(wid == 0, jnp.float32(-3e38), xmin + wf * sw)
    row_hi = jnp.where(wid == _NW - 1, jnp.float32(3e38),
                       xmin + (wf + 1.0) * sw)
    cand_lo = row_lo - wmax
    cand_hi = row_hi + wmax
    row_lo_v = jnp.full((_L,), row_lo)
    row_hi_v = jnp.full((_L,), row_hi)
    cand_lo_v = jnp.full((_L,), cand_lo)
    cand_hi_v = jnp.full((_L,), cand_hi)

    def sel_body(c, carry):
        nr, nc = carry
        xlv = xl[pl.ds(c * _L, _L)]
        iov = lane + c * _L
        mrow = (xlv >= row_lo_v) & (xlv < row_hi_v)
        mcand = (xlv >= cand_lo_v) & (xlv < cand_hi_v)
        plsc.store_compressed(rows.at[pl.ds(nr, _L)], iov, mask=mrow)
        plsc.store_compressed(cands.at[pl.ds(nc, _L)], iov, mask=mcand)
        nr = nr + _scalar_i32(plsc.all_reduce_population_count(mrow))
        nc = nc + _scalar_i32(plsc.all_reduce_population_count(mcand))
        return nr, nc

    nrow, ncand = lax.fori_loop(0, _NCHUNK, sel_body, (0, 0))

    sentinel = jnp.full((_L,), _NPAD - 1, jnp.int32)
    cands[pl.ds(ncand, _L)] = sentinel
    for k in range(6):
        rows[pl.ds(nrow + k * _L, _L)] = sentinel

    nchunk_c = (ncand + (_L - 1)) >> 4

    def compact_body(c, carry):
        off = c * _L
        idxv = cands[pl.ds(off, _L)]
        xsg = plsc.load_gather(xl, [idxv])
        ysg = plsc.load_gather(yl, [idxv])
        xeg = plsc.load_gather(xh, [idxv])
        yeg = plsc.load_gather(yh, [idxv])
        labg = plsc.load_gather(lab, [idxv])
        cxs[pl.ds(off, _L)] = xsg
        cys[pl.ds(off, _L)] = ysg
        cxe[pl.ds(off, _L)] = xeg
        cye[pl.ds(off, _L)] = yeg
        css[pl.ds(off, _L)] = plsc.load_gather(sc, [idxv])
        cpo[pl.ds(off, _L)] = (idxv << 3) | (labg & 7)
        car[pl.ds(off, _L)] = (xeg - xsg) * (yeg - ysg)
        return carry

    lax.fori_loop(0, nchunk_c, compact_body, 0)

    def _row_bcast(r):
        oiv = plsc.load_gather(rows, [jnp.full((_L,), r, jnp.int32)])
        xli = plsc.load_gather(xl, [oiv])
        yli = plsc.load_gather(yl, [oiv])
        xhi = plsc.load_gather(xh, [oiv])
        yhi = plsc.load_gather(yh, [oiv])
        sci = plsc.load_gather(sc, [oiv])
        labv = plsc.load_gather(lab, [oiv])
        poi = (oiv << 3) | (labv & 7)
        ari = (xhi - xli) * (yhi - yli)
        return xli, yli, xhi, yhi, sci, poi, ari

    def row_body(r2, carry_row):
        r0 = r2 * 2
        r1 = jnp.minimum(r0 + 1, nrow - 1)
        b0 = _row_bcast(r0)
        b1 = _row_bcast(r1)

        def chunk_body(cidx, accs):
            acc0, acc1 = accs
            off = cidx * _L
            xlj = cxs[pl.ds(off, _L)]
            ylj = cys[pl.ds(off, _L)]
            xhj = cxe[pl.ds(off, _L)]
            yhj = cye[pl.ds(off, _L)]
            scj = css[pl.ds(off, _L)]
            poj = cpo[pl.ds(off, _L)]
            arj = car[pl.ds(off, _L)]

            def one(b, acc):
                xli, yli, xhi, yhi, sci, poi, ari = b
                w = jnp.maximum(
                    jnp.minimum(xhi, xhj) - jnp.maximum(xli, xlj), 0.0)
                h = jnp.maximum(
                    jnp.minimum(yhi, yhj) - jnp.maximum(yli, ylj), 0.0)
                inter = w * h
                union = ari + arj - inter
                iou_hit = inter + inter > union
                same = ((poj ^ poi) & 7) == 0
                pri = (scj > sci) | ((scj == sci) & (poj > poi))
                return acc | (iou_hit & same & pri)

            return one(b0, acc0), one(b1, acc1)

        acc0 = jnp.zeros((_L,), dtype=jnp.bool_)
        acc0, acc1 = lax.fori_loop(0, nchunk_c, chunk_body, (acc0, acc0))
        out0 = jnp.where(jnp.full((_L,), jnp.any(acc0)), 0.0, b0[4])
        out1 = jnp.where(jnp.full((_L,), jnp.any(acc1)), 0.0, b1[4])
        plsc.store_scatter(outv, [jnp.full((_L,), r0, dtype=jnp.int32)],
                           out0, mask=lane0)
        plsc.store_scatter(outv, [jnp.full((_L,), r1, dtype=jnp.int32)],
                           out1, mask=lane0)
        return carry_row

    lax.fori_loop(0, (nrow + 1) >> 1, row_body, 0)

    def scat_body(c, carry):
        base = c * _OB
        for k in range(_OB // _L):
            myidx[pl.ds(k * _L, _L)] = rows[pl.ds(base + k * _L, _L)]
        pltpu.async_copy(outv.at[pl.ds(base, _OB)],
                         out_h.at[myidx], sem).wait()
        return carry

    lax.fori_loop(0, lax.div(nrow + _OB - 1, _OB), scat_body, 0)


_nms = functools.partial(
    pl.kernel,
    out_type=jax.ShapeDtypeStruct((_NPAD,), jnp.float32),
    mesh=plsc.VectorSubcoreMesh(core_axis_name="c", subcore_axis_name="s"),
    compiler_params=pltpu.CompilerParams(needs_layout_passes=False),
    scratch_types=[
        pltpu.VMEM((_NPAD,), jnp.float32),
        pltpu.VMEM((_NPAD,), jnp.float32),
        pltpu.VMEM((_NPAD,), jnp.float32),
        pltpu.VMEM((_NPAD,), jnp.float32),
        pltpu.VMEM((_NPAD,), jnp.float32),
        pltpu.VMEM((_NPAD,), jnp.int32),
        pltpu.VMEM((8,), jnp.float32),
        pltpu.VMEM((_CAP,), jnp.int32),
        pltpu.VMEM((_CAP,), jnp.int32),
        pltpu.VMEM((_CAP,), jnp.float32),
        pltpu.VMEM((_CAP,), jnp.float32),
        pltpu.VMEM((_CAP,), jnp.float32),
        pltpu.VMEM((_CAP,), jnp.float32),
        pltpu.VMEM((_CAP,), jnp.float32),
        pltpu.VMEM((_CAP,), jnp.int32),
        pltpu.VMEM((_CAP,), jnp.float32),
        pltpu.VMEM((_CAP,), jnp.float32),
        pltpu.VMEM((_OB,), jnp.int32),
        pltpu.SemaphoreType.DMA,
    ],
)(_nms_body)


def kernel(boxes, scores, pred_labels):
    xl = boxes[:, 0]
    yl = boxes[:, 1]
    xh = boxes[:, 2]
    yh = boxes[:, 3]
    xmin = jnp.min(xl)
    rng = jnp.maximum(jnp.max(xl) - xmin, jnp.float32(1e-30))
    wmax = jnp.max(xh - xl)
    par = jnp.stack([xmin, rng / _NW, wmax,
                     jnp.float32(0), jnp.float32(0), jnp.float32(0),
                     jnp.float32(0), jnp.float32(0)])

    npadf = jnp.zeros((_NPAD - _N,), jnp.float32)
    xl_p = jnp.concatenate([xl, jnp.full((_NPAD - _N,), _BIG, jnp.float32)])
    yl_p = jnp.concatenate([yl, npadf])
    xh_p = jnp.concatenate([xh, npadf])
    yh_p = jnp.concatenate([yh, npadf])
    sc_p = jnp.concatenate([scores, npadf])
    lab_p = jnp.concatenate([pred_labels.astype(jnp.int32),
                             jnp.full((_NPAD - _N,), -1, jnp.int32)])

    out = _nms(xl_p, yl_p, xh_p, yh_p, sc_p, lab_p, par)
    return out[:_N]


# speedup vs baseline: 1.0030x; 1.0030x over previous
import functools

import jax
import jax.numpy as jnp
from jax import lax
from jax.experimental import pallas as pl
from jax.experimental.pallas import tpu as pltpu
from jax.experimental.pallas import tpu_sc as plsc

_N = 5000
_L = 16
_NW = 32
_NPAD = 5120
_NCHUNK = _NPAD // _L
_CAP = _NPAD + 128
_OB = 80

_BIG = 1e30


def _scalar_i32(v):
    return jnp.max(v)


def _scalar_f32(v):
    return jnp.max(v)


def _nms_body(xl_h, yl_h, xh_h, yh_h, sc_h, lab_h, par_h, out_h,
              xl, yl, xh, yh, sc, lab, parv,
              cands, rows,
              cxs, cys, cxe, cye, css, cpo, car,
              outv, myidx, sem):
    cid = lax.axis_index("c")
    sid = lax.axis_index("s")
    wid = sid * 2 + cid
    pltpu.sync_copy(xl_h, xl)
    pltpu.sync_copy(yl_h, yl)
    pltpu.sync_copy(xh_h, xh)
    pltpu.sync_copy(yh_h, yh)
    pltpu.sync_copy(sc_h, sc)
    pltpu.sync_copy(lab_h, lab)
    pltpu.sync_copy(par_h, parv)

    lane = lax.iota(jnp.int32, _L)
    lane0 = lane == 0
    zero16 = jnp.zeros((_L,), jnp.int32)
    xmin = _scalar_f32(plsc.load_gather(parv, [zero16]))
    sw = _scalar_f32(plsc.load_gather(parv, [zero16 + 1]))
    wmax = _scalar_f32(plsc.load_gather(parv, [zero16 + 2]))

    wf = wid.astype(jnp.float32)
    row_lo = jnp.where(wid == 0, jnp.float32(-3e38), xmin + wf * sw)
    row_hi = jnp.where(wid == _NW - 1, jnp.float32(3e38),
                       xmin + (wf + 1.0) * sw)
    cand_lo = row_lo - wmax
    cand_hi = row_hi + wmax
    row_lo_v = jnp.full((_L,), row_lo)
    row_hi_v = jnp.full((_L,), row_hi)
    cand_lo_v = jnp.full((_L,), cand_lo)
    cand_hi_v = jnp.full((_L,), cand_hi)

    def sel_body(c, carry):
        nr, nc = carry
        xlv = xl[pl.ds(c * _L, _L)]
        iov = lane + c * _L
        mrow = (xlv >= row_lo_v) & (xlv < row_hi_v)
        mcand = (xlv >= cand_lo_v) & (xlv < cand_hi_v)
        plsc.store_compressed(rows.at[pl.ds(nr, _L)], iov, mask=mrow)
        plsc.store_compressed(cands.at[pl.ds(nc, _L)], iov, mask=mcand)
        nr = nr + _scalar_i32(plsc.all_reduce_population_count(mrow))
        nc = nc + _scalar_i32(plsc.all_reduce_population_count(mcand))
        return nr, nc

    nrow, ncand = lax.fori_loop(0, _NCHUNK, sel_body, (0, 0))

    sentinel = jnp.full((_L,), _NPAD - 1, jnp.int32)
    cands[pl.ds(ncand, _L)] = sentinel
    for k in range(6):
        rows[pl.ds(nrow + k * _L, _L)] = sentinel

    nchunk_c = (ncand + (_L - 1)) >> 4

    def compact_body(c, carry):
        off = c * _L
        idxv = cands[pl.ds(off, _L)]
        xsg = plsc.load_gather(xl, [idxv])
        ysg = plsc.load_gather(yl, [idxv])
        xeg = plsc.load_gather(xh, [idxv])
        yeg = plsc.load_gather(yh, [idxv])
        labg = plsc.load_gather(lab, [idxv])
        cxs[pl.ds(off, _L)] = xsg
        cys[pl.ds(off, _L)] = ysg
        cxe[pl.ds(off, _L)] = xeg
        cye[pl.ds(off, _L)] = yeg
        css[pl.ds(off, _L)] = plsc.load_gather(sc, [idxv])
        cpo[pl.ds(off, _L)] = (idxv << 3) | (labg & 7)
        car[pl.ds(off, _L)] = (xeg - xsg) * (yeg - ysg)
        return carry

    lax.fori_loop(0, nchunk_c, compact_body, 0)

    def _row_bcast(r):
        oiv = plsc.load_gather(rows, [jnp.full((_L,), r, jnp.int32)])
        xli = plsc.load_gather(xl, [oiv])
        yli = plsc.load_gather(yl, [oiv])
        xhi = plsc.load_gather(xh, [oiv])
        yhi = plsc.load_gather(yh, [oiv])
        sci = plsc.load_gather(sc, [oiv])
        labv = plsc.load_gather(lab, [oiv])
        poi = (oiv << 3) | (labv & 7)
        ari = (xhi - xli) * (yhi - yli)
        return xli, yli, xhi, yhi, sci, poi, ari

    def row_body(r2, carry_row):
        r0 = r2 * 2
        r1 = jnp.minimum(r0 + 1, nrow - 1)
        b0 = _row_bcast(r0)
        b1 = _row_bcast(r1)

        def chunk_body(cidx, accs):
            acc0, acc1 = accs
            off = cidx * _L
            xlj = cxs[pl.ds(off, _L)]
            ylj = cys[pl.ds(off, _L)]
            xhj = cxe[pl.ds(off, _L)]
            yhj = cye[pl.ds(off, _L)]
            scj = css[pl.ds(off, _L)]
            poj = cpo[pl.ds(off, _L)]
            arj = car[pl.ds(off, _L)]

            def one(b, acc):
                xli, yli, xhi, yhi, sci, poi, ari = b
                w = jnp.maximum(
                    jnp.minimum(xhi, xhj) - jnp.maximum(xli, xlj), 0.0)
                h = jnp.maximum(
                    jnp.minimum(yhi, yhj) - jnp.maximum(yli, ylj), 0.0)
                inter = w * h
                union = ari + arj - inter
                iou_hit = inter + inter > union
                same = ((poj ^ poi) & 7) == 0
                pri = (scj > sci) | ((scj == sci) & (poj > poi))
                return acc | (iou_hit & same & pri)

            return one(b0, acc0), one(b1, acc1)

        acc0 = jnp.zeros((_L,), dtype=jnp.bool_)
        acc0, acc1 = lax.fori_loop(0, nchunk_c, chunk_body, (acc0, acc0))
        out0 = jnp.where(jnp.full((_L,), jnp.any(acc0)), 0.0, b0[4])
        out1 = jnp.where(jnp.full((_L,), jnp.any(acc1)), 0.0, b1[4])
        plsc.store_scatter(outv, [jnp.full((_L,), r0, dtype=jnp.int32)],
                           out0, mask=lane0)
        plsc.store_scatter(outv, [jnp.full((_L,), r1, dtype=jnp.int32)],
                           out1, mask=lane0)
        return carry_row

    lax.fori_loop(0, (nrow + 1) >> 1, row_body, 0)

    def scat_body(c, carry):
        base = c * _OB
        for k in range(_OB // _L):
            myidx[pl.ds(k * _L, _L)] = rows[pl.ds(base + k * _L, _L)]
        pltpu.async_copy(outv.at[pl.ds(base, _OB)],
                         out_h.at[myidx], sem).wait()
        return carry

    lax.fori_loop(0, lax.div(nrow + _OB - 1, _OB), scat_body, 0)


_nms = functools.partial(
    pl.kernel,
    out_type=jax.ShapeDtypeStruct((_NPAD,), jnp.float32),
    mesh=plsc.VectorSubcoreMesh(core_axis_name="c", subcore_axis_name="s"),
    compiler_params=pltpu.CompilerParams(needs_layout_passes=False),
    scratch_types=[
        pltpu.VMEM((_NPAD,), jnp.float32),
        pltpu.VMEM((_NPAD,), jnp.float32),
        pltpu.VMEM((_NPAD,), jnp.float32),
        pltpu.VMEM((_NPAD,), jnp.float32),
        pltpu.VMEM((_NPAD,), jnp.float32),
        pltpu.VMEM((_NPAD,), jnp.int32),
        pltpu.VMEM((8,), jnp.float32),
        pltpu.VMEM((_CAP,), jnp.int32),
        pltpu.VMEM((_CAP,), jnp.int32),
        pltpu.VMEM((_CAP,), jnp.float32),
        pltpu.VMEM((_CAP,), jnp.float32),
        pltpu.VMEM((_CAP,), jnp.float32),
        pltpu.VMEM((_CAP,), jnp.float32),
        pltpu.VMEM((_CAP,), jnp.float32),
        pltpu.VMEM((_CAP,), jnp.int32),
        pltpu.VMEM((_CAP,), jnp.float32),
        pltpu.VMEM((_CAP,), jnp.float32),
        pltpu.VMEM((_OB,), jnp.int32),
        pltpu.SemaphoreType.DMA,
    ],
)(_nms_body)


def kernel(boxes, scores, pred_labels):
    xl = boxes[:, 0]
    yl = boxes[:, 1]
    xh = boxes[:, 2]
    yh = boxes[:, 3]
    xmin = jnp.min(xl)
    rng = jnp.maximum(jnp.max(xl) - xmin, jnp.float32(1e-30))
    wmax = jnp.max(xh - xl)
    par = jnp.stack([xmin, rng / _NW, wmax,
                     jnp.float32(0), jnp.float32(0), jnp.float32(0),
                     jnp.float32(0), jnp.float32(0)])

    npadf = jnp.zeros((_NPAD - _N,), jnp.float32)
    xl_p = jnp.concatenate([xl, jnp.full((_NPAD - _N,), jnp.nan, jnp.float32)])
    yl_p = jnp.concatenate([yl, npadf])
    xh_p = jnp.concatenate([xh, npadf])
    yh_p = jnp.concatenate([yh, npadf])
    sc_p = jnp.concatenate([scores, npadf])
    lab_p = jnp.concatenate([pred_labels.astype(jnp.int32),
                             jnp.full((_NPAD - _N,), -1, jnp.int32)])

    out = _nms(xl_p, yl_p, xh_p, yh_p, sc_p, lab_p, par)
    return out[:_N]


# speedup vs baseline: 2.4735x; 2.4661x over previous
import functools

import jax
import jax.numpy as jnp
from jax import lax
from jax.experimental import pallas as pl
from jax.experimental.pallas import tpu as pltpu
from jax.experimental.pallas import tpu_sc as plsc

_N = 5000
_L = 16
_NW = 32
_RPW = 160
_NPAD = _NW * _RPW
_CPW = _RPW // _L

_BIG = 1e30
_OB = 80
_OR = _RPW // _OB


def _nms_body(xl_h, yl_h, xh_h, yh_h, sc_h, lab_h, ord_h, ord3_h, lo_h, hi_h,
              out_h,
              xl, yl, xh, yh, sc, lab, ordv,
              xs, ys, xe, ye, ss, po, ar,
              lov, hiv, outv, myidx_a, myidx_b, sem):
    cid = lax.axis_index("c")
    sid = lax.axis_index("s")
    wid = sid * 2 + cid
    pltpu.sync_copy(xl_h, xl)
    pltpu.sync_copy(yl_h, yl)
    pltpu.sync_copy(xh_h, xh)
    pltpu.sync_copy(yh_h, yh)
    pltpu.sync_copy(sc_h, sc)
    pltpu.sync_copy(lab_h, lab)
    pltpu.sync_copy(ord_h, ordv)
    pltpu.sync_copy(lo_h, lov)
    pltpu.sync_copy(hi_h, hiv)

    lane = lax.iota(jnp.int32, _L)
    lane0 = lane == 0
    wsp = jnp.full((_L,), wid, dtype=jnp.int32)
    lo = jnp.max(plsc.load_gather(lov, [wsp]))
    hi = jnp.max(plsc.load_gather(hiv, [wsp]))
    clo = lo >> 4
    chi = (hi + (_L - 1)) >> 4
    base = wid * _RPW
    plo = jnp.minimum(clo, wid * _CPW)
    phi = jnp.maximum(chi, wid * _CPW + _CPW)

    def perm_body(c, carry):
        off = c * _L
        idxv = ordv[pl.ds(off, _L)]
        xsg = plsc.load_gather(xl, [idxv])
        ysg = plsc.load_gather(yl, [idxv])
        xeg = plsc.load_gather(xh, [idxv])
        yeg = plsc.load_gather(yh, [idxv])
        xs[pl.ds(off, _L)] = xsg
        ys[pl.ds(off, _L)] = ysg
        xe[pl.ds(off, _L)] = xeg
        ye[pl.ds(off, _L)] = yeg
        ss[pl.ds(off, _L)] = plsc.load_gather(sc, [idxv])
        labg = plsc.load_gather(lab, [idxv])
        po[pl.ds(off, _L)] = (idxv << 3) | (labg & 7)
        ar[pl.ds(off, _L)] = (xeg - xsg) * (yeg - ysg)
        return carry

    lax.fori_loop(plo, phi, perm_body, 0)

    def _row_bcast(i):
        isp = jnp.full((_L,), i, dtype=jnp.int32)
        return (plsc.load_gather(xs, [isp]), plsc.load_gather(ys, [isp]),
                plsc.load_gather(xe, [isp]), plsc.load_gather(ye, [isp]),
                plsc.load_gather(ss, [isp]), plsc.load_gather(po, [isp]),
                plsc.load_gather(ar, [isp]))

    def row_body(r2, carry_row):
        r0 = r2 * 2
        r1 = r0 + 1
        b0 = _row_bcast(base + r0)
        b1 = _row_bcast(base + r1)

        def chunk_body(cidx, accs):
            acc0, acc1 = accs
            off = cidx * _L
            xlj = xs[pl.ds(off, _L)]
            ylj = ys[pl.ds(off, _L)]
            xhj = xe[pl.ds(off, _L)]
            yhj = ye[pl.ds(off, _L)]
            scj = ss[pl.ds(off, _L)]
            poj = po[pl.ds(off, _L)]
            arj = ar[pl.ds(off, _L)]

            def one(b, acc):
                xli, yli, xhi, yhi, sci, poi, ari = b
                w = jnp.maximum(
                    jnp.minimum(xhi, xhj) - jnp.maximum(xli, xlj), 0.0)
                h = jnp.maximum(
                    jnp.minimum(yhi, yhj) - jnp.maximum(yli, ylj), 0.0)
                inter = w * h
                union = ari + arj - inter
                iou_hit = inter + inter > union
                same = ((poj ^ poi) & 7) == 0
                pri = (scj > sci) | ((scj == sci) & (poj > poi))
                return acc | (iou_hit & same & pri)

            return one(b0, acc0), one(b1, acc1)

        acc0 = jnp.zeros((_L,), dtype=jnp.bool_)
        acc0, acc1 = lax.fori_loop(clo, chi, chunk_body, (acc0, acc0))
        out0 = jnp.where(jnp.full((_L,), jnp.any(acc0)), 0.0, b0[4])
        out1 = jnp.where(jnp.full((_L,), jnp.any(acc1)), 0.0, b1[4])
        plsc.store_scatter(outv, [jnp.full((_L,), r0, dtype=jnp.int32)],
                           out0, mask=lane0)
        plsc.store_scatter(outv, [jnp.full((_L,), r1, dtype=jnp.int32)],
                           out1, mask=lane0)
        return carry_row

    lax.fori_loop(0, _RPW // 2, row_body, 0)
    pltpu.sync_copy(ord3_h.at[wid, 0], myidx_a)
    pltpu.sync_copy(ord3_h.at[wid, 1], myidx_b)
    pltpu.async_copy(outv.at[pl.ds(0, _OB)], out_h.at[myidx_a], sem).wait()
    pltpu.async_copy(outv.at[pl.ds(_OB, _OB)], out_h.at[myidx_b], sem).wait()


_nms = functools.partial(
    pl.kernel,
    out_type=jax.ShapeDtypeStruct((_NPAD,), jnp.float32),
    mesh=plsc.VectorSubcoreMesh(core_axis_name="c", subcore_axis_name="s"),
    compiler_params=pltpu.CompilerParams(needs_layout_passes=False),
    scratch_types=[
        pltpu.VMEM((_NPAD,), jnp.float32),
        pltpu.VMEM((_NPAD,), jnp.float32),
        pltpu.VMEM((_NPAD,), jnp.float32),
        pltpu.VMEM((_NPAD,), jnp.float32),
        pltpu.VMEM((_NPAD,), jnp.float32),
        pltpu.VMEM((_NPAD,), jnp.int32),
        pltpu.VMEM((_NPAD,), jnp.int32),
        pltpu.VMEM((_NPAD,), jnp.float32),
        pltpu.VMEM((_NPAD,), jnp.float32),
        pltpu.VMEM((_NPAD,), jnp.float32),
        pltpu.VMEM((_NPAD,), jnp.float32),
        pltpu.VMEM((_NPAD,), jnp.float32),
        pltpu.VMEM((_NPAD,), jnp.int32),
        pltpu.VMEM((_NPAD,), jnp.float32),
        pltpu.VMEM((_NW,), jnp.int32),
        pltpu.VMEM((_NW,), jnp.int32),
        pltpu.VMEM((_RPW,), jnp.float32),
        pltpu.VMEM((_OB,), jnp.int32),
        pltpu.VMEM((_OB,), jnp.int32),
        pltpu.SemaphoreType.DMA,
    ],
)(_nms_body)


def kernel(boxes, scores, pred_labels):
    xl = boxes[:, 0]
    yl = boxes[:, 1]
    xh = boxes[:, 2]
    yh = boxes[:, 3]
    wmax = jnp.max(xh - xl)

    iota = jnp.arange(_N, dtype=jnp.int32)
    xls, order = lax.sort((xl, iota), num_keys=1, is_stable=False)

    firsts = xls[0::_RPW]
    lasts = jnp.concatenate([xls[_RPW - 1::_RPW], xls[_N - 1:]])
    lo_arr = jnp.searchsorted(xls, firsts - wmax, side="left").astype(jnp.int32)
    hi_arr = jnp.searchsorted(xls, lasts + wmax, side="right").astype(jnp.int32)

    npadf = jnp.zeros((_NPAD - _N,), jnp.float32)
    xl_p = jnp.concatenate([xl, jnp.full((_NPAD - _N,), _BIG, jnp.float32)])
    yl_p = jnp.concatenate([yl, npadf])
    xh_p = jnp.concatenate([xh, npadf])
    yh_p = jnp.concatenate([yh, npadf])
    sc_p = jnp.concatenate([scores, npadf])
    lab_p = jnp.concatenate([pred_labels.astype(jnp.int32),
                             jnp.full((_NPAD - _N,), -1, jnp.int32)])
    ord_p = jnp.concatenate([order,
                             jnp.arange(_N, _NPAD, dtype=jnp.int32)])

    out = _nms(xl_p, yl_p, xh_p, yh_p, sc_p, lab_p,
               ord_p, ord_p.reshape(_NW, _OR, _OB), lo_arr, hi_arr)
    return out[:_N]
